# Initial kernel scaffold; baseline (speedup 1.0000x reference)
import jax
import jax.numpy as jnp
from jax.experimental import pallas as pl


def kernel(texts, table, W, b):
    raise NotImplementedError("write your pallas kernel here")


# speedup vs baseline: 1.4898x; 1.4898x over previous
import functools
import math

import jax
import jax.numpy as jnp
from jax import lax
from jax.experimental import pallas as pl
from jax.experimental.pallas import tpu as pltpu
from jax.experimental.pallas import tpu_sc as plsc

_B = 16384
_L = 50
_LP = 52
_D = 64
_C = 3
_NC = 2
_NS = 16
_NW = _NC * _NS
_BAGS_W = _B // _NW
_BAGS_CHUNK = 2
_IDX_CHUNK = _BAGS_CHUNK * _LP
_CHUNKS = _BAGS_W // _BAGS_CHUNK
_SCALE = math.sqrt(_D) / _L


def _sc_body(texts_hbm, table_hbm, w_hbm, b_hbm, out_hbm,
             idx_v, rows_v, w_v, b_v, out_v, gsem):
    wid = lax.axis_index("s") * _NC + lax.axis_index("c")
    bag_base = wid * _BAGS_W
    idx_base = bag_base * _LP

    pltpu.sync_copy(w_hbm, w_v)
    pltpu.sync_copy(b_hbm, b_v)

    wv = tuple(tuple(w_v[c, pl.ds(k * 16, 16)] * _SCALE for k in range(4))
               for c in range(_C))
    bvec = b_v[pl.ds(0, 16)]
    lane = jnp.arange(16, dtype=jnp.int32)
    lane_ok = lane < _C

    def chunk_body(g, carry):
        off = idx_base + g * _IDX_CHUNK
        pltpu.sync_copy(texts_hbm.at[pl.ds(off, _IDX_CHUNK)], idx_v)
        pltpu.async_copy(table_hbm.at[idx_v], rows_v, gsem).wait()
        for bb in range(_BAGS_CHUNK):
            def row_body(j, acc):
                r = bb * _LP + j
                return tuple(acc[k] + rows_v[r, pl.ds(k * 16, 16)]
                             for k in range(4))
            acc = lax.fori_loop(
                0, _L, row_body,
                tuple(jnp.zeros((16,), jnp.float32) for _ in range(4)))
            bag = g * _BAGS_CHUNK + bb
            s = []
            for c in range(_C):
                t = acc[0] * wv[c][0]
                for k in range(1, 4):
                    t = t + acc[k] * wv[c][k]
                s.append(jnp.sum(t))
            outvec = jnp.where(
                lane == 0, s[0],
                jnp.where(lane == 1, s[1],
                          jnp.where(lane == 2, s[2], 0.0))) + bvec
            plsc.store_scatter(out_v, [bag * _C + lane], outvec,
                               mask=lane_ok)
        return carry

    lax.fori_loop(0, _CHUNKS, chunk_body, 0)
    pltpu.sync_copy(out_v, out_hbm.at[pl.ds(bag_base * _C, _BAGS_W * _C)])


@jax.jit
def _run(texts_flat, table, w, b16):
    mesh = plsc.VectorSubcoreMesh(core_axis_name="c", subcore_axis_name="s")
    return pl.kernel(
        _sc_body,
        out_type=jax.ShapeDtypeStruct((_B * _C,), jnp.float32),
        mesh=mesh,
        scratch_types=[
            pltpu.VMEM((_IDX_CHUNK,), jnp.int32),
            pltpu.VMEM((_IDX_CHUNK, _D), jnp.float32),
            pltpu.VMEM((_C, _D), jnp.float32),
            pltpu.VMEM((16,), jnp.float32),
            pltpu.VMEM((_BAGS_W * _C,), jnp.float32),
            pltpu.SemaphoreType.DMA,
        ],
        compiler_params=pltpu.CompilerParams(
            needs_layout_passes=False, use_tc_tiling_on_sc=False),
    )(texts_flat, table, w, b16)


def kernel(texts, table, W, b):
    texts_p = jnp.pad(texts, ((0, 0), (0, _LP - _L)))
    texts_flat = texts_p.reshape(-1)
    b16 = jnp.zeros((16,), b.dtype).at[:_C].set(b)
    return _run(texts_flat, table, W, b16).reshape(_B, _C)
